# baseline (speedup 1.0000x reference)
import functools

import numpy as np
import jax
import jax.numpy as jnp
from jax import lax
from jax.experimental import pallas as pl
from jax.experimental.pallas import tpu as pltpu
from jax.experimental.pallas import tpu_sc as plsc

B, S, D = 1024, 200, 64
N = B * S
NW = 32
RPT = N // NW
C = 400
NCHUNK = RPT // C
G = 80
NG = C // G
NV = D // 16
SCALE = 8.0


def _pos_np():
    pos = np.arange(S)[:, None].astype(np.float64)
    i = np.arange(D)[None, :].astype(np.float64)
    angle_rates = 1.0 / np.power(10000, 2 * (i // 2) / np.float32(D))
    ang = pos * angle_rates
    ang[:, 0::2] = np.sin(ang[:, 0::2])
    ang[:, 1::2] = np.cos(ang[:, 1::2])
    return ang.astype(np.float32)


_POS = _pos_np()

_mesh = plsc.VectorSubcoreMesh(core_axis_name="c", subcore_axis_name="s")


@functools.partial(
    pl.kernel,
    out_type=jax.ShapeDtypeStruct((N, D), jnp.float32),
    mesh=_mesh,
    scratch_types=[
        pltpu.VMEM((C,), jnp.int32),
        pltpu.VMEM((C, D), jnp.float32),
        pltpu.VMEM((S, D), jnp.float32),
        pltpu.SemaphoreType.DMA,
    ],
    compiler_params=pltpu.CompilerParams(use_tc_tiling_on_sc=False),
)
def _emb_kernel(idx_hbm, w_hbm, pos_hbm, out_hbm, idx_v, rows_v, pos_v, sem):
    wid = lax.axis_index("s") * 2 + lax.axis_index("c")
    base = wid * RPT
    pltpu.sync_copy(pos_hbm, pos_v)

    def chunk_body(c, carry):
        row0 = base + c * C
        pltpu.sync_copy(idx_hbm.at[pl.ds(row0, C)], idx_v)
        copies = [
            pltpu.async_copy(w_hbm.at[idx_v.at[pl.ds(j * G, G)]],
                             rows_v.at[pl.ds(j * G, G)], sem)
            for j in range(NG)
        ]
        for cp in copies:
            cp.wait()

        def p_body(p, carry2):
            pv = [pos_v[p, pl.ds(d * 16, 16)] for d in range(NV)]
            for r in range(C // S):
                i = r * S + p
                for d in range(NV):
                    rows_v[i, pl.ds(d * 16, 16)] = (
                        rows_v[i, pl.ds(d * 16, 16)] * SCALE + pv[d])
            return carry2

        lax.fori_loop(0, S, p_body, 0)
        pltpu.sync_copy(rows_v, out_hbm.at[pl.ds(row0, C)])
        return carry

    lax.fori_loop(0, NCHUNK, chunk_body, 0)


def kernel(x, W):
    idx = x.reshape(-1).astype(jnp.int32)
    pos = jnp.asarray(_POS)
    out = _emb_kernel(idx, W, pos)
    return out.reshape(B, S, D)


# speedup vs baseline: 1.0447x; 1.0447x over previous
import functools

import numpy as np
import jax
import jax.numpy as jnp
from jax import lax
from jax.experimental import pallas as pl
from jax.experimental.pallas import tpu as pltpu
from jax.experimental.pallas import tpu_sc as plsc

B, S, D = 1024, 200, 64
N = B * S
NW = 32
RPT = N // NW
C = 400
NCHUNK = RPT // C
G = 80
NG = C // G
NV = D // 16
NBUF = 3
SCALE = 8.0


def _pos_np():
    pos = np.arange(S)[:, None].astype(np.float64)
    i = np.arange(D)[None, :].astype(np.float64)
    angle_rates = 1.0 / np.power(10000, 2 * (i // 2) / np.float32(D))
    ang = pos * angle_rates
    ang[:, 0::2] = np.sin(ang[:, 0::2])
    ang[:, 1::2] = np.cos(ang[:, 1::2])
    return ang.astype(np.float32)


_POS = _pos_np()

_mesh = plsc.VectorSubcoreMesh(core_axis_name="c", subcore_axis_name="s")


@functools.partial(
    pl.kernel,
    out_type=jax.ShapeDtypeStruct((N, D), jnp.float32),
    mesh=_mesh,
    scratch_types=[
        pltpu.VMEM((RPT,), jnp.int32),
        pltpu.VMEM((S, D), jnp.float32),
    ] + [pltpu.VMEM((C, D), jnp.float32) for _ in range(NBUF)]
      + [pltpu.SemaphoreType.DMA for _ in range(2 * NBUF)],
    compiler_params=pltpu.CompilerParams(use_tc_tiling_on_sc=False),
)
def _emb_kernel(idx_hbm, w_hbm, pos_hbm, out_hbm, idx_v, pos_v,
                rows0, rows1, rows2, g0, g1, g2, w0, w1, w2):
    rows = (rows0, rows1, rows2)
    gsem = (g0, g1, g2)
    wsem = (w0, w1, w2)
    wid = lax.axis_index("s") * 2 + lax.axis_index("c")
    base = wid * RPT
    pltpu.sync_copy(idx_hbm.at[pl.ds(base, RPT)], idx_v)
    pltpu.sync_copy(pos_hbm, pos_v)

    def fire_gathers(c):
        buf = c % NBUF
        return [
            pltpu.async_copy(
                w_hbm.at[idx_v.at[pl.ds(c * C + j * G, G)]],
                rows[buf].at[pl.ds(j * G, G)], gsem[buf])
            for j in range(NG)
        ]

    gathers = {0: fire_gathers(0)}
    writebacks = {}
    for c in range(NCHUNK):
        buf = c % NBUF
        if c + 1 < NCHUNK:
            if c - 2 >= 0:
                writebacks.pop(c - 2).wait()
            gathers[c + 1] = fire_gathers(c + 1)
        for cp in gathers.pop(c):
            cp.wait()

        rbuf = rows[buf]

        @plsc.parallel_loop(0, S, step=1, unroll=4)
        def p_body(p):
            pv = [pos_v[p, pl.ds(d * 16, 16)] for d in range(NV)]
            for r in range(C // S):
                i = r * S + p
                for d in range(NV):
                    rbuf[i, pl.ds(d * 16, 16)] = (
                        rbuf[i, pl.ds(d * 16, 16)] * SCALE + pv[d])

        writebacks[c] = pltpu.async_copy(
            rbuf, out_hbm.at[pl.ds(base + c * C, C)], wsem[buf])
    for c in sorted(writebacks):
        writebacks.pop(c).wait()


def kernel(x, W):
    idx = x.reshape(-1).astype(jnp.int32)
    pos = jnp.asarray(_POS)
    out = _emb_kernel(idx, W, pos)
    return out.reshape(B, S, D)
